# Initial kernel scaffold; baseline (speedup 1.0000x reference)
import jax
import jax.numpy as jnp
from jax.experimental import pallas as pl


def kernel(prev_relation, current_entity, actions_id, queries, emb, W_ih, W_hh, b_ih, b_hh, W1, b1, W2, b2):
    raise NotImplementedError("write your pallas kernel here")


# speedup vs baseline: 3.8553x; 3.8553x over previous
import jax
import jax.numpy as jnp
from jax import lax
from jax.experimental import pallas as pl
from jax.experimental.pallas import tpu as pltpu
from jax.experimental.pallas import tpu_sc as plsc

D = 128
B = 1024
A = 200
SE = 256
AE = 256
H = 512

NC, NS = 2, 16
NW = NC * NS
BPW = B // NW
NCH = 5
CH = (2 * A) // NCH
SW = 208

_F32 = jnp.float32


def _mesh():
    return plsc.VectorSubcoreMesh(
        core_axis_name="c", subcore_axis_name="s",
        num_cores=NC, num_subcores=NS)


def _wid():
    return lax.axis_index("s") * NC + lax.axis_index("c")


def _gather3_body(prev_hbm, cur_hbm, qry_hbm, emb_hbm,
                  rel_out, cur_out, qry_out,
                  ip, ic, iq, rp, rc, rq, sem):
    base = _wid() * BPW
    pltpu.sync_copy(prev_hbm.at[pl.ds(base, BPW)], ip)
    pltpu.sync_copy(cur_hbm.at[pl.ds(base, BPW)], ic)
    pltpu.sync_copy(qry_hbm.at[pl.ds(base, BPW)], iq)
    cp1 = pltpu.async_copy(emb_hbm.at[ip], rp, sem)
    cp2 = pltpu.async_copy(emb_hbm.at[ic], rc, sem)
    cp3 = pltpu.async_copy(emb_hbm.at[iq], rq, sem)
    cp1.wait()
    cp2.wait()
    cp3.wait()
    pltpu.sync_copy(rp, rel_out.at[pl.ds(base, BPW)])
    pltpu.sync_copy(rc, cur_out.at[pl.ds(base, BPW)])
    pltpu.sync_copy(rq, qry_out.at[pl.ds(base, BPW)])


def _gather3(prev, cur, qry, emb):
    f = pl.kernel(
        _gather3_body,
        out_type=[jax.ShapeDtypeStruct((B, D), _F32)] * 3,
        mesh=_mesh(),
        scratch_types=[
            pltpu.VMEM((BPW,), jnp.int32),
            pltpu.VMEM((BPW,), jnp.int32),
            pltpu.VMEM((BPW,), jnp.int32),
            pltpu.VMEM((BPW, D), _F32),
            pltpu.VMEM((BPW, D), _F32),
            pltpu.VMEM((BPW, D), _F32),
            pltpu.SemaphoreType.DMA,
        ],
    )
    return f(prev, cur, qry, emb)


def _dense_body(rel_ref, cur_ref, qry_ref, wih_ref, bih_ref, bhh_ref,
                w1_ref, b1_ref, w2_ref, b2_ref, out_ref):
    def dot_t(a, w):
        return lax.dot_general(
            a, w, (((1,), (1,)), ((), ())),
            precision=lax.Precision.HIGHEST, preferred_element_type=_F32)

    rel = rel_ref[...]
    cur = cur_ref[...]
    qry = qry_ref[...]
    wih = wih_ref[...]
    gates = dot_t(rel, wih[:, :D]) + dot_t(cur, wih[:, D:]) \
        + bih_ref[...] + bhh_ref[...]
    i_g = jax.nn.sigmoid(gates[:, :SE])
    g_g = jnp.tanh(gates[:, 2 * SE:3 * SE])
    o_g = jax.nn.sigmoid(gates[:, 3 * SE:])
    h = o_g * jnp.tanh(i_g * g_g)
    w1 = w1_ref[...]
    hid = dot_t(h, w1[:, :SE]) + dot_t(qry, w1[:, SE:SE + D]) \
        + dot_t(cur, w1[:, SE + D:]) + b1_ref[...]
    hid = jnp.maximum(hid, 0.0)
    out = dot_t(hid, w2_ref[...]) + b2_ref[...]
    out_ref[...] = jnp.maximum(out, 0.0)


def _dense(rel_e, cur_e, qry_e, W_ih, b_ih, b_hh, W1, b1, W2, b2):
    return pl.pallas_call(
        _dense_body,
        out_shape=jax.ShapeDtypeStruct((B, AE), _F32),
    )(rel_e, cur_e, qry_e, W_ih, b_ih.reshape(1, 4 * SE),
      b_hh.reshape(1, 4 * SE), W1, b1.reshape(1, H), W2, b2.reshape(1, AE))


def _score_body(idx_hbm, out_hbm, emb_hbm, scores_hbm,
                idx_all, o_all, rows, scores_v, sem):
    wid = _wid()
    base = wid * BPW
    pltpu.sync_copy(idx_hbm.at[pl.ds(wid * (BPW * NCH), BPW * NCH)], idx_all)
    pltpu.sync_copy(out_hbm.at[pl.ds(base, BPW)], o_all)

    lane15 = lax.iota(jnp.int32, 16) == 15

    def per_b(bi, carry):
        cps = [
            pltpu.async_copy(emb_hbm.at[idx_all.at[bi * NCH + j]],
                             rows.at[pl.ds(j * CH, CH)], sem)
            for j in range(NCH)
        ]
        for cp in cps:
            cp.wait()
        oc = [o_all[bi, pl.ds(c * 16, 16)] for c in range(16)]
        bi_v = jnp.full((16,), bi, jnp.int32)

        def per_a(a, c2):
            acc = rows[2 * a, pl.ds(0, 16)] * oc[0]
            for c in range(1, 16):
                r = 2 * a + (c // 8)
                acc = acc + rows[r, pl.ds((c % 8) * 16, 16)] * oc[c]
            ps = plsc.cumsum(acc)
            plsc.store_scatter(scores_v, [bi_v, jnp.full((16,), a, jnp.int32)],
                               ps, mask=lane15)
            return c2

        lax.fori_loop(0, A, per_a, 0)
        return carry

    lax.fori_loop(0, BPW, per_b, 0)
    pltpu.sync_copy(scores_v, scores_hbm.at[pl.ds(base, BPW)])


def _score(idx_rs, out, emb):
    f = pl.kernel(
        _score_body,
        out_type=jax.ShapeDtypeStruct((B, SW), _F32),
        mesh=_mesh(),
        scratch_types=[
            pltpu.VMEM((BPW * NCH, CH), jnp.int32),
            pltpu.VMEM((BPW, AE), _F32),
            pltpu.VMEM((2 * A, D), _F32),
            pltpu.VMEM((BPW, SW), _F32),
            pltpu.SemaphoreType.DMA,
        ],
        compiler_params=pltpu.CompilerParams(needs_layout_passes=False),
    )
    return f(idx_rs, out, emb)


def _lsm_body(s_ref, ids_ref, out_ref):
    s = s_ref[...]
    ids = ids_ref[...]
    col = lax.broadcasted_iota(jnp.int32, (B, SW), 1)
    s = jnp.where(ids == 0, -99999.0, s)
    s = jnp.where(col >= A, -jnp.inf, s)
    m = jnp.max(s, axis=-1, keepdims=True)
    e = jnp.exp(s - m)
    out_ref[...] = s - m - jnp.log(jnp.sum(e, axis=-1, keepdims=True))


def _lsm(scores, ids_pad):
    return pl.pallas_call(
        _lsm_body,
        out_shape=jax.ShapeDtypeStruct((B, SW), _F32),
    )(scores, ids_pad)


def kernel(prev_relation, current_entity, actions_id, queries, emb,
           W_ih, W_hh, b_ih, b_hh, W1, b1, W2, b2):
    del W_hh
    prev32 = prev_relation.astype(jnp.int32)
    cur32 = current_entity.astype(jnp.int32)
    qry32 = queries.astype(jnp.int32)
    act32 = actions_id.astype(jnp.int32)
    idx_rs = act32.reshape(B * NCH, CH)

    rel_e, cur_e, qry_e = _gather3(prev32, cur32, qry32, emb)
    out = _dense(rel_e, cur_e, qry_e, W_ih, b_ih, b_hh, W1, b1, W2, b2)
    scores = _score(idx_rs, out, emb)

    ent_ids = act32[:, :, 1]
    ids_pad = jnp.pad(ent_ids, ((0, 0), (0, SW - A)))
    logits = _lsm(scores, ids_pad)[:, :A]
    return (logits, actions_id[:, :, 0], actions_id[:, :, 1])


# speedup vs baseline: 4.7757x; 1.2387x over previous
import jax
import jax.numpy as jnp
from jax import lax
from jax.experimental import pallas as pl
from jax.experimental.pallas import tpu as pltpu
from jax.experimental.pallas import tpu_sc as plsc

D = 128
B = 1024
A = 200
SE = 256
AE = 256
H = 512

NC, NS = 2, 16
NW = NC * NS
BPW = B // NW
NH = 2
AH = A // NH
NCH = 5
CH = (2 * AH) // NCH
SW = 208

_F32 = jnp.float32


def _mesh():
    return plsc.VectorSubcoreMesh(
        core_axis_name="c", subcore_axis_name="s",
        num_cores=NC, num_subcores=NS)


def _wid():
    return lax.axis_index("s") * NC + lax.axis_index("c")


def _gather3_body(prev_hbm, cur_hbm, qry_hbm, emb_hbm,
                  rel_out, cur_out, qry_out,
                  ip, ic, iq, rp, rc, rq, sem):
    base = _wid() * BPW
    pltpu.sync_copy(prev_hbm.at[pl.ds(base, BPW)], ip)
    pltpu.sync_copy(cur_hbm.at[pl.ds(base, BPW)], ic)
    pltpu.sync_copy(qry_hbm.at[pl.ds(base, BPW)], iq)
    cp1 = pltpu.async_copy(emb_hbm.at[ip], rp, sem)
    cp2 = pltpu.async_copy(emb_hbm.at[ic], rc, sem)
    cp3 = pltpu.async_copy(emb_hbm.at[iq], rq, sem)
    cp1.wait()
    cp2.wait()
    cp3.wait()
    pltpu.sync_copy(rp, rel_out.at[pl.ds(base, BPW)])
    pltpu.sync_copy(rc, cur_out.at[pl.ds(base, BPW)])
    pltpu.sync_copy(rq, qry_out.at[pl.ds(base, BPW)])


def _gather3(prev, cur, qry, emb):
    f = pl.kernel(
        _gather3_body,
        out_type=[jax.ShapeDtypeStruct((B, D), _F32)] * 3,
        mesh=_mesh(),
        scratch_types=[
            pltpu.VMEM((BPW,), jnp.int32),
            pltpu.VMEM((BPW,), jnp.int32),
            pltpu.VMEM((BPW,), jnp.int32),
            pltpu.VMEM((BPW, D), _F32),
            pltpu.VMEM((BPW, D), _F32),
            pltpu.VMEM((BPW, D), _F32),
            pltpu.SemaphoreType.DMA,
        ],
    )
    return f(prev, cur, qry, emb)


def _dense_body(rel_ref, cur_ref, qry_ref, wih_ref, bih_ref, bhh_ref,
                w1_ref, b1_ref, w2_ref, b2_ref, out_ref):
    def dot_t(a, w):
        return lax.dot_general(
            a, w, (((1,), (1,)), ((), ())),
            precision=lax.Precision.HIGHEST, preferred_element_type=_F32)

    rel = rel_ref[...]
    cur = cur_ref[...]
    qry = qry_ref[...]
    wih = wih_ref[...]
    gates = dot_t(rel, wih[:, :D]) + dot_t(cur, wih[:, D:]) \
        + bih_ref[...] + bhh_ref[...]
    i_g = jax.nn.sigmoid(gates[:, :SE])
    g_g = jnp.tanh(gates[:, 2 * SE:3 * SE])
    o_g = jax.nn.sigmoid(gates[:, 3 * SE:])
    h = o_g * jnp.tanh(i_g * g_g)
    w1 = w1_ref[...]
    hid = dot_t(h, w1[:, :SE]) + dot_t(qry, w1[:, SE:SE + D]) \
        + dot_t(cur, w1[:, SE + D:]) + b1_ref[...]
    hid = jnp.maximum(hid, 0.0)
    out = dot_t(hid, w2_ref[...]) + b2_ref[...]
    out_ref[...] = jnp.maximum(out, 0.0)


def _dense(rel_e, cur_e, qry_e, W_ih, b_ih, b_hh, W1, b1, W2, b2):
    return pl.pallas_call(
        _dense_body,
        out_shape=jax.ShapeDtypeStruct((B, AE), _F32),
    )(rel_e, cur_e, qry_e, W_ih, b_ih.reshape(1, 4 * SE),
      b_hh.reshape(1, 4 * SE), W1, b1.reshape(1, H), W2, b2.reshape(1, AE))


def _score_body(idx_hbm, out_hbm, emb_hbm, scores_hbm,
                idx_all, o_all, rows0, rows1, scores_v, sem0, sem1):
    wid = _wid()
    base = wid * BPW
    pltpu.sync_copy(
        idx_hbm.at[pl.ds(wid * (BPW * NH * NCH), BPW * NH * NCH)], idx_all)
    pltpu.sync_copy(out_hbm.at[pl.ds(base, BPW)], o_all)

    lane15 = lax.iota(jnp.int32, 16) == 15

    def issue(u, rows, sem):
        for j in range(NCH):
            pltpu.async_copy(emb_hbm.at[idx_all.at[u * NCH + j]],
                             rows.at[pl.ds(j * CH, CH)], sem)

    def drain(rows, sem):
        pltpu.make_async_copy(emb_hbm.at[pl.ds(0, 2 * AH)], rows, sem).wait()

    def compute(bi, half, rows):
        oc = [o_all[bi, pl.ds(c * 16, 16)] for c in range(16)]
        bi_v = jnp.full((16,), bi, jnp.int32)

        def per_a(al, c2):
            acc = rows[2 * al, pl.ds(0, 16)] * oc[0]
            for c in range(1, 16):
                r = 2 * al + (c // 8)
                acc = acc + rows[r, pl.ds((c % 8) * 16, 16)] * oc[c]
            ps = plsc.cumsum(acc)
            plsc.store_scatter(
                scores_v,
                [bi_v, jnp.full((16,), al + half * AH, jnp.int32)],
                ps, mask=lane15)
            return c2

        lax.fori_loop(0, AH, per_a, 0)

    issue(0, rows0, sem0)

    def step(t, carry):
        issue(2 * t + 1, rows1, sem1)
        drain(rows0, sem0)
        compute(t, 0, rows0)

        @pl.when(t + 1 < BPW)
        def _():
            issue(2 * t + 2, rows0, sem0)

        drain(rows1, sem1)
        compute(t, 1, rows1)
        return carry

    lax.fori_loop(0, BPW, step, 0)
    pltpu.sync_copy(scores_v, scores_hbm.at[pl.ds(base, BPW)])


def _score(idx_rs, out, emb):
    f = pl.kernel(
        _score_body,
        out_type=jax.ShapeDtypeStruct((B, SW), _F32),
        mesh=_mesh(),
        scratch_types=[
            pltpu.VMEM((BPW * NH * NCH, CH), jnp.int32),
            pltpu.VMEM((BPW, AE), _F32),
            pltpu.VMEM((2 * AH, D), _F32),
            pltpu.VMEM((2 * AH, D), _F32),
            pltpu.VMEM((BPW, SW), _F32),
            pltpu.SemaphoreType.DMA,
            pltpu.SemaphoreType.DMA,
        ],
        compiler_params=pltpu.CompilerParams(needs_layout_passes=False),
    )
    return f(idx_rs, out, emb)


def _lsm_body(s_ref, ids_ref, out_ref):
    s = s_ref[...]
    ids = ids_ref[...]
    col = lax.broadcasted_iota(jnp.int32, (B, SW), 1)
    s = jnp.where(ids == 0, -99999.0, s)
    s = jnp.where(col >= A, -jnp.inf, s)
    m = jnp.max(s, axis=-1, keepdims=True)
    e = jnp.exp(s - m)
    out_ref[...] = s - m - jnp.log(jnp.sum(e, axis=-1, keepdims=True))


def _lsm(scores, ids_pad):
    return pl.pallas_call(
        _lsm_body,
        out_shape=jax.ShapeDtypeStruct((B, SW), _F32),
    )(scores, ids_pad)


def kernel(prev_relation, current_entity, actions_id, queries, emb,
           W_ih, W_hh, b_ih, b_hh, W1, b1, W2, b2):
    del W_hh
    prev32 = prev_relation.astype(jnp.int32)
    cur32 = current_entity.astype(jnp.int32)
    qry32 = queries.astype(jnp.int32)
    act32 = actions_id.astype(jnp.int32)
    idx_rs = act32.reshape(B * NH * NCH, CH)

    rel_e, cur_e, qry_e = _gather3(prev32, cur32, qry32, emb)
    out = _dense(rel_e, cur_e, qry_e, W_ih, b_ih, b_hh, W1, b1, W2, b2)
    scores = _score(idx_rs, out, emb)

    ent_ids = act32[:, :, 1]
    ids_pad = jnp.pad(ent_ids, ((0, 0), (0, SW - A)))
    logits = _lsm(scores, ids_pad)[:, :A]
    return (logits, actions_id[:, :, 0], actions_id[:, :, 1])


# speedup vs baseline: 5.1619x; 1.0809x over previous
import jax
import jax.numpy as jnp
from jax import lax
from jax.experimental import pallas as pl
from jax.experimental.pallas import tpu as pltpu
from jax.experimental.pallas import tpu_sc as plsc

D = 128
B = 1024
A = 200
SE = 256
AE = 256
H = 512

NC, NS = 2, 16
NW = NC * NS
BPW = B // NW
NH = 2
AH = A // NH
NCH = 5
CH = (2 * AH) // NCH
SW = 208

_F32 = jnp.float32


def _mesh():
    return plsc.VectorSubcoreMesh(
        core_axis_name="c", subcore_axis_name="s",
        num_cores=NC, num_subcores=NS)


def _wid():
    return lax.axis_index("s") * NC + lax.axis_index("c")


def _gather3_body(prev_hbm, cur_hbm, qry_hbm, emb_hbm,
                  rel_out, cur_out, qry_out,
                  ip, ic, iq, rp, rc, rq, sem):
    base = _wid() * BPW
    pltpu.sync_copy(prev_hbm.at[pl.ds(base, BPW)], ip)
    pltpu.sync_copy(cur_hbm.at[pl.ds(base, BPW)], ic)
    pltpu.sync_copy(qry_hbm.at[pl.ds(base, BPW)], iq)
    cp1 = pltpu.async_copy(emb_hbm.at[ip], rp, sem)
    cp2 = pltpu.async_copy(emb_hbm.at[ic], rc, sem)
    cp3 = pltpu.async_copy(emb_hbm.at[iq], rq, sem)
    cp1.wait()
    cp2.wait()
    cp3.wait()
    pltpu.sync_copy(rp, rel_out.at[pl.ds(base, BPW)])
    pltpu.sync_copy(rc, cur_out.at[pl.ds(base, BPW)])
    pltpu.sync_copy(rq, qry_out.at[pl.ds(base, BPW)])


def _gather3(prev, cur, qry, emb):
    f = pl.kernel(
        _gather3_body,
        out_type=[jax.ShapeDtypeStruct((B, D), _F32)] * 3,
        mesh=_mesh(),
        scratch_types=[
            pltpu.VMEM((BPW,), jnp.int32),
            pltpu.VMEM((BPW,), jnp.int32),
            pltpu.VMEM((BPW,), jnp.int32),
            pltpu.VMEM((BPW, D), _F32),
            pltpu.VMEM((BPW, D), _F32),
            pltpu.VMEM((BPW, D), _F32),
            pltpu.SemaphoreType.DMA,
        ],
    )
    return f(prev, cur, qry, emb)


def _dense_body(rel_ref, cur_ref, qry_ref, wih_ref, bih_ref, bhh_ref,
                w1_ref, b1_ref, w2_ref, b2_ref, out_ref):
    def dot_t(a, w):
        return lax.dot_general(
            a, w, (((1,), (1,)), ((), ())),
            precision=lax.Precision.HIGHEST, preferred_element_type=_F32)

    rel = rel_ref[...]
    cur = cur_ref[...]
    qry = qry_ref[...]
    wih = wih_ref[...]
    gates = dot_t(rel, wih[:, :D]) + dot_t(cur, wih[:, D:]) \
        + bih_ref[...] + bhh_ref[...]
    i_g = jax.nn.sigmoid(gates[:, :SE])
    g_g = jnp.tanh(gates[:, 2 * SE:3 * SE])
    o_g = jax.nn.sigmoid(gates[:, 3 * SE:])
    h = o_g * jnp.tanh(i_g * g_g)
    w1 = w1_ref[...]
    hid = dot_t(h, w1[:, :SE]) + dot_t(qry, w1[:, SE:SE + D]) \
        + dot_t(cur, w1[:, SE + D:]) + b1_ref[...]
    hid = jnp.maximum(hid, 0.0)
    out = dot_t(hid, w2_ref[...]) + b2_ref[...]
    out_ref[...] = jnp.maximum(out, 0.0)


def _dense(rel_e, cur_e, qry_e, W_ih, b_ih, b_hh, W1, b1, W2, b2):
    return pl.pallas_call(
        _dense_body,
        out_shape=jax.ShapeDtypeStruct((B, AE), _F32),
    )(rel_e, cur_e, qry_e, W_ih, b_ih.reshape(1, 4 * SE),
      b_hh.reshape(1, 4 * SE), W1, b1.reshape(1, H), W2, b2.reshape(1, AE))


def _score_body(idx_hbm, out_hbm, emb_hbm, scores_hbm,
                idx_all, o_all, rows0, rows1, scores_v, sem0, sem1):
    wid = _wid()
    base = wid * BPW
    pltpu.sync_copy(
        idx_hbm.at[pl.ds(wid * (BPW * NH * NCH), BPW * NH * NCH)], idx_all)
    pltpu.sync_copy(out_hbm.at[pl.ds(base, BPW)], o_all)

    lane15 = lax.iota(jnp.int32, 16) == 15

    def issue(u, rows, sem):
        for j in range(NCH):
            pltpu.async_copy(emb_hbm.at[idx_all.at[u * NCH + j]],
                             rows.at[pl.ds(j * CH, CH)], sem)

    def drain(rows, sem):
        pltpu.make_async_copy(emb_hbm.at[pl.ds(0, 2 * AH)], rows, sem).wait()

    def compute(bi, half, rows):
        oc = [o_all[bi, pl.ds(c * 16, 16)] for c in range(16)]

        def per_a2(i, fiv):
            for u in range(2):
                prods = [
                    rows[4 * i + 2 * u + (c // 8), pl.ds((c % 8) * 16, 16)]
                    * oc[c]
                    for c in range(16)
                ]
                while len(prods) > 1:
                    prods = [prods[k] + prods[k + 1]
                             for k in range(0, len(prods), 2)]
                ps = plsc.cumsum(prods[0])
                plsc.store_scatter(scores_v, [fiv], ps, mask=lane15)
                fiv = fiv + 1
            return fiv

        fiv0 = jnp.full((16,), bi * SW + half * AH, jnp.int32)
        lax.fori_loop(0, AH // 2, per_a2, fiv0)

    issue(0, rows0, sem0)

    def step(t, carry):
        issue(2 * t + 1, rows1, sem1)
        drain(rows0, sem0)
        compute(t, 0, rows0)

        @pl.when(t + 1 < BPW)
        def _():
            issue(2 * t + 2, rows0, sem0)

        drain(rows1, sem1)
        compute(t, 1, rows1)
        return carry

    lax.fori_loop(0, BPW, step, 0)
    pltpu.sync_copy(scores_v, scores_hbm.at[pl.ds(base * SW, BPW * SW)])


def _score(idx_rs, out, emb):
    f = pl.kernel(
        _score_body,
        out_type=jax.ShapeDtypeStruct((B * SW,), _F32),
        mesh=_mesh(),
        scratch_types=[
            pltpu.VMEM((BPW * NH * NCH, CH), jnp.int32),
            pltpu.VMEM((BPW, AE), _F32),
            pltpu.VMEM((2 * AH, D), _F32),
            pltpu.VMEM((2 * AH, D), _F32),
            pltpu.VMEM((BPW * SW,), _F32),
            pltpu.SemaphoreType.DMA,
            pltpu.SemaphoreType.DMA,
        ],
        compiler_params=pltpu.CompilerParams(needs_layout_passes=False),
    )
    return f(idx_rs, out, emb)


def _lsm_body(s_ref, ids_ref, out_ref):
    s = s_ref[...]
    ids = ids_ref[...]
    col = lax.broadcasted_iota(jnp.int32, (B, SW), 1)
    s = jnp.where(ids == 0, -99999.0, s)
    s = jnp.where(col >= A, -jnp.inf, s)
    m = jnp.max(s, axis=-1, keepdims=True)
    e = jnp.exp(s - m)
    out_ref[...] = s - m - jnp.log(jnp.sum(e, axis=-1, keepdims=True))


def _lsm(scores, ids_pad):
    return pl.pallas_call(
        _lsm_body,
        out_shape=jax.ShapeDtypeStruct((B, SW), _F32),
    )(scores, ids_pad)


def kernel(prev_relation, current_entity, actions_id, queries, emb,
           W_ih, W_hh, b_ih, b_hh, W1, b1, W2, b2):
    del W_hh
    prev32 = prev_relation.astype(jnp.int32)
    cur32 = current_entity.astype(jnp.int32)
    qry32 = queries.astype(jnp.int32)
    act32 = actions_id.astype(jnp.int32)
    idx_rs = act32.reshape(B * NH * NCH, CH)

    rel_e, cur_e, qry_e = _gather3(prev32, cur32, qry32, emb)
    out = _dense(rel_e, cur_e, qry_e, W_ih, b_ih, b_hh, W1, b1, W2, b2)
    scores = _score(idx_rs, out, emb).reshape(B, SW)

    ent_ids = act32[:, :, 1]
    ids_pad = jnp.pad(ent_ids, ((0, 0), (0, SW - A)))
    logits = _lsm(scores, ids_pad)[:, :A]
    return (logits, actions_id[:, :, 0], actions_id[:, :, 1])


# speedup vs baseline: 6.4985x; 1.2589x over previous
import jax
import jax.numpy as jnp
from jax import lax
from jax.experimental import pallas as pl
from jax.experimental.pallas import tpu as pltpu
from jax.experimental.pallas import tpu_sc as plsc

D = 128
B = 1024
A = 200
SE = 256
AE = 256
H = 512

NC, NS = 2, 16
NW = NC * NS
BPW = B // NW
NH = 2
AH = A // NH
NCH = 5
CH = (2 * AH) // NCH
SW = 208

_F32 = jnp.float32


def _mesh():
    return plsc.VectorSubcoreMesh(
        core_axis_name="c", subcore_axis_name="s",
        num_cores=NC, num_subcores=NS)


def _wid():
    return lax.axis_index("s") * NC + lax.axis_index("c")


def _gather3_body(prev_hbm, cur_hbm, qry_hbm, emb_hbm,
                  rel_out, cur_out, qry_out,
                  ip, ic, iq, rp, rc, rq, sem):
    base = _wid() * BPW
    pltpu.sync_copy(prev_hbm.at[pl.ds(base, BPW)], ip)
    pltpu.sync_copy(cur_hbm.at[pl.ds(base, BPW)], ic)
    pltpu.sync_copy(qry_hbm.at[pl.ds(base, BPW)], iq)
    cp1 = pltpu.async_copy(emb_hbm.at[ip], rp, sem)
    cp2 = pltpu.async_copy(emb_hbm.at[ic], rc, sem)
    cp3 = pltpu.async_copy(emb_hbm.at[iq], rq, sem)
    cp1.wait()
    cp2.wait()
    cp3.wait()
    pltpu.sync_copy(rp, rel_out.at[pl.ds(base, BPW)])
    pltpu.sync_copy(rc, cur_out.at[pl.ds(base, BPW)])
    pltpu.sync_copy(rq, qry_out.at[pl.ds(base, BPW)])


def _gather3(prev, cur, qry, emb):
    f = pl.kernel(
        _gather3_body,
        out_type=[jax.ShapeDtypeStruct((B, D), _F32)] * 3,
        mesh=_mesh(),
        scratch_types=[
            pltpu.VMEM((BPW,), jnp.int32),
            pltpu.VMEM((BPW,), jnp.int32),
            pltpu.VMEM((BPW,), jnp.int32),
            pltpu.VMEM((BPW, D), _F32),
            pltpu.VMEM((BPW, D), _F32),
            pltpu.VMEM((BPW, D), _F32),
            pltpu.SemaphoreType.DMA,
        ],
    )
    return f(prev, cur, qry, emb)


def _dense_body(rel_ref, cur_ref, qry_ref, wih_ref, bih_ref, bhh_ref,
                w1_ref, b1_ref, w2_ref, b2_ref, out_ref):
    def dot_t(a, w):
        return lax.dot_general(
            a, w, (((1,), (1,)), ((), ())),
            precision=lax.Precision.HIGHEST, preferred_element_type=_F32)

    rel = rel_ref[...]
    cur = cur_ref[...]
    qry = qry_ref[...]
    wih = wih_ref[...]
    gates = dot_t(rel, wih[:, :D]) + dot_t(cur, wih[:, D:]) \
        + bih_ref[...] + bhh_ref[...]
    i_g = jax.nn.sigmoid(gates[:, :SE])
    g_g = jnp.tanh(gates[:, 2 * SE:3 * SE])
    o_g = jax.nn.sigmoid(gates[:, 3 * SE:])
    h = o_g * jnp.tanh(i_g * g_g)
    w1 = w1_ref[...]
    hid = dot_t(h, w1[:, :SE]) + dot_t(qry, w1[:, SE:SE + D]) \
        + dot_t(cur, w1[:, SE + D:]) + b1_ref[...]
    hid = jnp.maximum(hid, 0.0)
    out = dot_t(hid, w2_ref[...]) + b2_ref[...]
    out_ref[...] = jnp.maximum(out, 0.0)


def _dense(rel_e, cur_e, qry_e, W_ih, b_ih, b_hh, W1, b1, W2, b2):
    return pl.pallas_call(
        _dense_body,
        out_shape=jax.ShapeDtypeStruct((B, AE), _F32),
    )(rel_e, cur_e, qry_e, W_ih, b_ih.reshape(1, 4 * SE),
      b_hh.reshape(1, 4 * SE), W1, b1.reshape(1, H), W2, b2.reshape(1, AE))


def _score_body(idx_hbm, out_hbm, emb_hbm, scores_hbm,
                idx_all, o_all, rows0, rows1, scores_v, sem0, sem1):
    wid = _wid()
    base = wid * BPW
    pltpu.sync_copy(
        idx_hbm.at[pl.ds(wid * (BPW * NH * NCH), BPW * NH * NCH)], idx_all)
    pltpu.sync_copy(out_hbm.at[pl.ds(base, BPW)], o_all)

    lane15 = lax.iota(jnp.int32, 16) == 15

    def issue(u, rows, sem):
        for j in range(NCH):
            pltpu.async_copy(emb_hbm.at[idx_all.at[u * NCH + j]],
                             rows.at[pl.ds(j * CH, CH)], sem)

    def drain(rows, sem):
        pltpu.make_async_copy(emb_hbm.at[pl.ds(0, 2 * AH)], rows, sem).wait()

    def compute(bi, half, rows):
        oc = [o_all[bi, pl.ds(c * 16, 16)] for c in range(16)]

        base_v = jnp.full((16,), bi * SW + half * AH, jnp.int32)

        @plsc.parallel_loop(0, AH, step=1, unroll=2)
        def _loop(al):
            prods = [
                rows[2 * al + (c // 8), pl.ds((c % 8) * 16, 16)] * oc[c]
                for c in range(16)
            ]
            while len(prods) > 1:
                prods = [prods[k] + prods[k + 1]
                         for k in range(0, len(prods), 2)]
            ps = plsc.cumsum(prods[0])
            plsc.store_scatter(scores_v, [base_v + al], ps, mask=lane15)

    issue(0, rows0, sem0)

    def step(t, carry):
        issue(2 * t + 1, rows1, sem1)
        drain(rows0, sem0)
        compute(t, 0, rows0)

        @pl.when(t + 1 < BPW)
        def _():
            issue(2 * t + 2, rows0, sem0)

        drain(rows1, sem1)
        compute(t, 1, rows1)
        return carry

    lax.fori_loop(0, BPW, step, 0)
    pltpu.sync_copy(scores_v, scores_hbm.at[pl.ds(base * SW, BPW * SW)])


def _score(idx_rs, out, emb):
    f = pl.kernel(
        _score_body,
        out_type=jax.ShapeDtypeStruct((B * SW,), _F32),
        mesh=_mesh(),
        scratch_types=[
            pltpu.VMEM((BPW * NH * NCH, CH), jnp.int32),
            pltpu.VMEM((BPW, AE), _F32),
            pltpu.VMEM((2 * AH, D), _F32),
            pltpu.VMEM((2 * AH, D), _F32),
            pltpu.VMEM((BPW * SW,), _F32),
            pltpu.SemaphoreType.DMA,
            pltpu.SemaphoreType.DMA,
        ],
        compiler_params=pltpu.CompilerParams(needs_layout_passes=False),
    )
    return f(idx_rs, out, emb)


def _lsm_body(s_ref, ids_ref, out_ref):
    s = s_ref[...]
    ids = ids_ref[...]
    col = lax.broadcasted_iota(jnp.int32, (B, SW), 1)
    s = jnp.where(ids == 0, -99999.0, s)
    s = jnp.where(col >= A, -jnp.inf, s)
    m = jnp.max(s, axis=-1, keepdims=True)
    e = jnp.exp(s - m)
    out_ref[...] = s - m - jnp.log(jnp.sum(e, axis=-1, keepdims=True))


def _lsm(scores, ids_pad):
    return pl.pallas_call(
        _lsm_body,
        out_shape=jax.ShapeDtypeStruct((B, SW), _F32),
    )(scores, ids_pad)


def kernel(prev_relation, current_entity, actions_id, queries, emb,
           W_ih, W_hh, b_ih, b_hh, W1, b1, W2, b2):
    del W_hh
    prev32 = prev_relation.astype(jnp.int32)
    cur32 = current_entity.astype(jnp.int32)
    qry32 = queries.astype(jnp.int32)
    act32 = actions_id.astype(jnp.int32)
    idx_rs = act32.reshape(B * NH * NCH, CH)

    rel_e, cur_e, qry_e = _gather3(prev32, cur32, qry32, emb)
    out = _dense(rel_e, cur_e, qry_e, W_ih, b_ih, b_hh, W1, b1, W2, b2)
    scores = _score(idx_rs, out, emb).reshape(B, SW)

    ent_ids = act32[:, :, 1]
    ids_pad = jnp.pad(ent_ids, ((0, 0), (0, SW - A)))
    logits = _lsm(scores, ids_pad)[:, :A]
    return (logits, actions_id[:, :, 0], actions_id[:, :, 1])


# speedup vs baseline: 6.5815x; 1.0128x over previous
import jax
import jax.numpy as jnp
from jax import lax
from jax.experimental import pallas as pl
from jax.experimental.pallas import tpu as pltpu
from jax.experimental.pallas import tpu_sc as plsc

D = 128
B = 1024
A = 200
SE = 256
AE = 256
H = 512

NC, NS = 2, 16
NW = NC * NS
BPW = B // NW
NH = 2
AH = A // NH
NCH = 5
CH = (2 * AH) // NCH
SW = 208

_F32 = jnp.float32


def _mesh():
    return plsc.VectorSubcoreMesh(
        core_axis_name="c", subcore_axis_name="s",
        num_cores=NC, num_subcores=NS)


def _wid():
    return lax.axis_index("s") * NC + lax.axis_index("c")


def _gather3_body(prev_hbm, cur_hbm, qry_hbm, emb_hbm,
                  rel_out, cur_out, qry_out,
                  ip, ic, iq, rp, rc, rq, sem):
    base = _wid() * BPW
    pltpu.sync_copy(prev_hbm.at[pl.ds(base, BPW)], ip)
    pltpu.sync_copy(cur_hbm.at[pl.ds(base, BPW)], ic)
    pltpu.sync_copy(qry_hbm.at[pl.ds(base, BPW)], iq)
    cp1 = pltpu.async_copy(emb_hbm.at[ip], rp, sem)
    cp2 = pltpu.async_copy(emb_hbm.at[ic], rc, sem)
    cp3 = pltpu.async_copy(emb_hbm.at[iq], rq, sem)
    cp1.wait()
    cp2.wait()
    cp3.wait()
    pltpu.sync_copy(rp, rel_out.at[pl.ds(base, BPW)])
    pltpu.sync_copy(rc, cur_out.at[pl.ds(base, BPW)])
    pltpu.sync_copy(rq, qry_out.at[pl.ds(base, BPW)])


def _gather3(prev, cur, qry, emb):
    f = pl.kernel(
        _gather3_body,
        out_type=[jax.ShapeDtypeStruct((B, D), _F32)] * 3,
        mesh=_mesh(),
        scratch_types=[
            pltpu.VMEM((BPW,), jnp.int32),
            pltpu.VMEM((BPW,), jnp.int32),
            pltpu.VMEM((BPW,), jnp.int32),
            pltpu.VMEM((BPW, D), _F32),
            pltpu.VMEM((BPW, D), _F32),
            pltpu.VMEM((BPW, D), _F32),
            pltpu.SemaphoreType.DMA,
        ],
    )
    return f(prev, cur, qry, emb)


def _dense_body(rel_ref, cur_ref, qry_ref, wih_ref, bih_ref, bhh_ref,
                w1_ref, b1_ref, w2_ref, b2_ref, out_ref):
    def dot_t(a, w):
        return lax.dot_general(
            a, w, (((1,), (1,)), ((), ())),
            precision=lax.Precision.HIGHEST, preferred_element_type=_F32)

    rel = rel_ref[...]
    cur = cur_ref[...]
    qry = qry_ref[...]
    wih = wih_ref[...]
    gates = dot_t(rel, wih[:, :D]) + dot_t(cur, wih[:, D:]) \
        + bih_ref[...] + bhh_ref[...]
    i_g = jax.nn.sigmoid(gates[:, :SE])
    g_g = jnp.tanh(gates[:, 2 * SE:3 * SE])
    o_g = jax.nn.sigmoid(gates[:, 3 * SE:])
    h = o_g * jnp.tanh(i_g * g_g)
    w1 = w1_ref[...]
    hid = dot_t(h, w1[:, :SE]) + dot_t(qry, w1[:, SE:SE + D]) \
        + dot_t(cur, w1[:, SE + D:]) + b1_ref[...]
    hid = jnp.maximum(hid, 0.0)
    out = dot_t(hid, w2_ref[...]) + b2_ref[...]
    out_ref[...] = jnp.maximum(out, 0.0)


def _dense(rel_e, cur_e, qry_e, W_ih, b_ih, b_hh, W1, b1, W2, b2):
    return pl.pallas_call(
        _dense_body,
        out_shape=jax.ShapeDtypeStruct((B, AE), _F32),
    )(rel_e, cur_e, qry_e, W_ih, b_ih.reshape(1, 4 * SE),
      b_hh.reshape(1, 4 * SE), W1, b1.reshape(1, H), W2, b2.reshape(1, AE))


def _score_body(idx_hbm, out_hbm, emb_hbm, scores_hbm,
                idx_all, o_all, rows0, rows1, scores_v, sem0, sem1):
    wid = _wid()
    base = wid * BPW
    pltpu.sync_copy(
        idx_hbm.at[pl.ds(wid * (BPW * NH * NCH), BPW * NH * NCH)], idx_all)
    pltpu.sync_copy(out_hbm.at[pl.ds(base, BPW)], o_all)

    lane15 = lax.iota(jnp.int32, 16) == 15

    def issue(u, rows, sem):
        for j in range(NCH):
            pltpu.async_copy(emb_hbm.at[idx_all.at[u * NCH + j]],
                             rows.at[pl.ds(j * CH, CH)], sem)

    def drain(rows, sem):
        pltpu.make_async_copy(emb_hbm.at[pl.ds(0, 2 * AH)], rows, sem).wait()

    def compute(bi, half, rows):
        oc = [o_all[bi, pl.ds(c * 16, 16)] for c in range(16)]

        base_v = jnp.full((16,), bi * SW + half * AH, jnp.int32)

        @plsc.parallel_loop(0, AH, step=1, unroll=2)
        def _loop(al):
            prods = [
                rows[2 * al + (c // 8), pl.ds((c % 8) * 16, 16)] * oc[c]
                for c in range(16)
            ]
            while len(prods) > 1:
                prods = [prods[k] + prods[k + 1]
                         for k in range(0, len(prods), 2)]
            ps = plsc.cumsum(prods[0])
            plsc.store_scatter(scores_v, [base_v + al], ps, mask=lane15)

    NG = SW // 16
    lane_i = lax.iota(jnp.int32, 16)
    lane_i2 = lane_i * 2
    LN2 = 0.6931471805599453

    def bcast_last_max(v):
        return plsc.cummax(lax.rev(plsc.cummax(v), (0,)))

    def log_softmax_b(bi):
        sv = []
        for g in range(NG):
            s = scores_v[pl.ds(bi * SW + g * 16, 16)]
            pvec = lane_i2 + (32 * g + 1)
            rowv = bi * (NH * NCH) + pvec // CH
            ids = plsc.load_gather(idx_all, [rowv, pvec % CH])
            s = jnp.where(ids == 0, -99999.0, s)
            if g == NG - 1:
                s = jnp.where(lane_i >= 8, -jnp.inf, s)
            sv.append(s)
        mt = sv
        while len(mt) > 1:
            mt = [jnp.maximum(mt[k], mt[k + 1]) if k + 1 < len(mt) else mt[k]
                  for k in range(0, len(mt), 2)]
        bmax = bcast_last_max(mt[0])
        ev = [jnp.exp(s - bmax) for s in sv]
        while len(ev) > 1:
            ev = [ev[k] + ev[k + 1] if k + 1 < len(ev) else ev[k]
                  for k in range(0, len(ev), 2)]
        bsum = bcast_last_max(plsc.cumsum(ev[0]))
        bits = plsc.bitcast(bsum, jnp.int32)
        e_val = ((bits >> 23) & 0xFF) - 127
        mant = plsc.bitcast((bits & 0x7FFFFF) | 0x3F800000, _F32)
        tq = (mant - 1.0) / (mant + 1.0)
        t2 = tq * tq
        lnm = 2.0 * tq * (1.0 + t2 * (1.0 / 3.0 + t2 * (0.2 + t2 / 7.0)))
        lsum = e_val.astype(_F32) * LN2 + lnm
        off = bmax + lsum
        for g in range(NG):
            sg = sv[g] - off
            scores_v[pl.ds(bi * SW + g * 16, 16)] = sg

    issue(0, rows0, sem0)

    def step(t, carry):
        issue(2 * t + 1, rows1, sem1)
        drain(rows0, sem0)
        compute(t, 0, rows0)

        @pl.when(t + 1 < BPW)
        def _():
            issue(2 * t + 2, rows0, sem0)

        drain(rows1, sem1)
        compute(t, 1, rows1)
        log_softmax_b(t)
        return carry

    lax.fori_loop(0, BPW, step, 0)
    pltpu.sync_copy(scores_v, scores_hbm.at[pl.ds(base * SW, BPW * SW)])


def _score(idx_rs, out, emb):
    f = pl.kernel(
        _score_body,
        out_type=jax.ShapeDtypeStruct((B * SW,), _F32),
        mesh=_mesh(),
        scratch_types=[
            pltpu.VMEM((BPW * NH * NCH, CH), jnp.int32),
            pltpu.VMEM((BPW, AE), _F32),
            pltpu.VMEM((2 * AH, D), _F32),
            pltpu.VMEM((2 * AH, D), _F32),
            pltpu.VMEM((BPW * SW,), _F32),
            pltpu.SemaphoreType.DMA,
            pltpu.SemaphoreType.DMA,
        ],
        compiler_params=pltpu.CompilerParams(needs_layout_passes=False),
    )
    return f(idx_rs, out, emb)


def kernel(prev_relation, current_entity, actions_id, queries, emb,
           W_ih, W_hh, b_ih, b_hh, W1, b1, W2, b2):
    del W_hh
    prev32 = prev_relation.astype(jnp.int32)
    cur32 = current_entity.astype(jnp.int32)
    qry32 = queries.astype(jnp.int32)
    act32 = actions_id.astype(jnp.int32)
    idx_rs = act32.reshape(B * NH * NCH, CH)

    rel_e, cur_e, qry_e = _gather3(prev32, cur32, qry32, emb)
    out = _dense(rel_e, cur_e, qry_e, W_ih, b_ih, b_hh, W1, b1, W2, b2)
    logits = _score(idx_rs, out, emb).reshape(B, SW)[:, :A]
    return (logits, actions_id[:, :, 0], actions_id[:, :, 1])


# speedup vs baseline: 11.6167x; 1.7651x over previous
import jax
import jax.numpy as jnp
from jax import lax
from jax.experimental import pallas as pl
from jax.experimental.pallas import tpu as pltpu
from jax.experimental.pallas import tpu_sc as plsc

D = 128
B = 1024
A = 200
SE = 256
AE = 256
H = 512

NC, NS = 2, 16
NW = NC * NS
BPW = B // NW
SW = 208

_F32 = jnp.float32


def _mesh():
    return plsc.VectorSubcoreMesh(
        core_axis_name="c", subcore_axis_name="s",
        num_cores=NC, num_subcores=NS)


def _wid():
    return lax.axis_index("s") * NC + lax.axis_index("c")


def _gather3_body(prev_hbm, cur_hbm, qry_hbm, emb_hbm,
                  rel_out, cur_out, qry_out,
                  ip, ic, iq, rp, rc, rq, sem):
    base = _wid() * BPW
    pltpu.sync_copy(prev_hbm.at[pl.ds(base, BPW)], ip)
    pltpu.sync_copy(cur_hbm.at[pl.ds(base, BPW)], ic)
    pltpu.sync_copy(qry_hbm.at[pl.ds(base, BPW)], iq)
    cp1 = pltpu.async_copy(emb_hbm.at[ip], rp, sem)
    cp2 = pltpu.async_copy(emb_hbm.at[ic], rc, sem)
    cp3 = pltpu.async_copy(emb_hbm.at[iq], rq, sem)
    cp1.wait()
    cp2.wait()
    cp3.wait()
    pltpu.sync_copy(rp, rel_out.at[pl.ds(base, BPW)])
    pltpu.sync_copy(rc, cur_out.at[pl.ds(base, BPW)])
    pltpu.sync_copy(rq, qry_out.at[pl.ds(base, BPW)])


def _gather3(prev, cur, qry, emb):
    f = pl.kernel(
        _gather3_body,
        out_type=[jax.ShapeDtypeStruct((B, D), _F32)] * 3,
        mesh=_mesh(),
        scratch_types=[
            pltpu.VMEM((BPW,), jnp.int32),
            pltpu.VMEM((BPW,), jnp.int32),
            pltpu.VMEM((BPW,), jnp.int32),
            pltpu.VMEM((BPW, D), _F32),
            pltpu.VMEM((BPW, D), _F32),
            pltpu.VMEM((BPW, D), _F32),
            pltpu.SemaphoreType.DMA,
        ],
    )
    return f(prev, cur, qry, emb)


def _dense_body(rel_ref, cur_ref, qry_ref, wih_ref, bih_ref, bhh_ref,
                w1_ref, b1_ref, w2_ref, b2_ref, out_ref):
    def dot_t(a, w):
        return lax.dot_general(
            a, w, (((1,), (1,)), ((), ())),
            preferred_element_type=_F32)

    rel = rel_ref[...]
    cur = cur_ref[...]
    qry = qry_ref[...]
    wih = wih_ref[...]
    gates = dot_t(rel, wih[:, :D]) + dot_t(cur, wih[:, D:]) \
        + bih_ref[...] + bhh_ref[...]
    i_g = jax.nn.sigmoid(gates[:, :SE])
    g_g = jnp.tanh(gates[:, 2 * SE:3 * SE])
    o_g = jax.nn.sigmoid(gates[:, 3 * SE:])
    h = o_g * jnp.tanh(i_g * g_g)
    w1 = w1_ref[...]
    hid = dot_t(h, w1[:, :SE]) + dot_t(qry, w1[:, SE:SE + D]) \
        + dot_t(cur, w1[:, SE + D:]) + b1_ref[...]
    hid = jnp.maximum(hid, 0.0)
    out = dot_t(hid, w2_ref[...]) + b2_ref[...]
    out_ref[...] = jnp.maximum(out, 0.0)


def _dense(rel_e, cur_e, qry_e, W_ih, b_ih, b_hh, W1, b1, W2, b2):
    return pl.pallas_call(
        _dense_body,
        out_shape=jax.ShapeDtypeStruct((B, AE), _F32),
    )(rel_e, cur_e, qry_e, W_ih, b_ih.reshape(1, 4 * SE),
      b_hh.reshape(1, 4 * SE), W1, b1.reshape(1, H), W2, b2.reshape(1, AE))


H_OFF = (0, 96)
H_N = (96, 104)
H_CHUNK = (((0, 0, 40), (1, 0, 40), (2, 0, 16)),
           ((2, 16, 24), (3, 0, 40), (4, 0, 40)))


def _score_body(rel_hbm, ent_hbm, out_hbm, emb_hbm, scores_hbm,
                idx_r, idx_e, o_all,
                rows_r0, rows_e0, rows_r1, rows_e1, scores_v, sem0, sem1):
    wid = _wid()
    base = wid * BPW
    nrow = BPW * 5
    pltpu.sync_copy(rel_hbm.at[pl.ds(wid * nrow, nrow)], idx_r)
    pltpu.sync_copy(ent_hbm.at[pl.ds(wid * nrow, nrow)], idx_e)
    pltpu.sync_copy(out_hbm.at[pl.ds(base, BPW)], o_all)

    lane15 = lax.iota(jnp.int32, 16) == 15

    def issue(bi, half, rows_r, rows_e, sem):
        dst = 0
        for (j, col, sz) in H_CHUNK[half]:
            pltpu.async_copy(emb_hbm.at[idx_r.at[bi * 5 + j, pl.ds(col, sz)]],
                             rows_r.at[pl.ds(dst, sz)], sem)
            pltpu.async_copy(emb_hbm.at[idx_e.at[bi * 5 + j, pl.ds(col, sz)]],
                             rows_e.at[pl.ds(dst, sz)], sem)
            dst += sz

    def drain(half, rows_r, rows_e, sem):
        n = H_N[half]
        pltpu.make_async_copy(
            emb_hbm.at[pl.ds(0, n)], rows_r.at[pl.ds(0, n)], sem).wait()
        pltpu.make_async_copy(
            emb_hbm.at[pl.ds(0, n)], rows_e.at[pl.ds(0, n)], sem).wait()

    def compute(bi, half, rows_r, rows_e):
        oc = [o_all[bi, pl.ds(c * 16, 16)] for c in range(16)]
        base_v = jnp.full((16,), bi * SW + H_OFF[half], jnp.int32)

        @plsc.parallel_loop(0, H_N[half], step=1, unroll=4)
        def _loop(al):
            prods = [rows_r[al, pl.ds(c * 16, 16)] * oc[c] for c in range(8)]
            prods += [rows_e[al, pl.ds((c - 8) * 16, 16)] * oc[c]
                      for c in range(8, 16)]
            while len(prods) > 1:
                prods = [prods[k] + prods[k + 1]
                         for k in range(0, len(prods), 2)]
            ps = plsc.cumsum(prods[0])
            plsc.store_scatter(scores_v, [base_v + al], ps, mask=lane15)

    NG = SW // 16
    lane_i = lax.iota(jnp.int32, 16)
    LN2 = 0.6931471805599453

    def bcast_last_max(v):
        return plsc.cummax(lax.rev(plsc.cummax(v), (0,)))

    def log_softmax_b(bi):
        sv = []
        for g in range(NG):
            s = scores_v[pl.ds(bi * SW + g * 16, 16)]
            p0 = g * 16
            r0 = p0 // 40
            brk = (r0 + 1) * 40 - p0
            if g == NG - 1:
                pv = jnp.minimum(lane_i + p0, A - 1)
                rowc = jnp.full((16,), bi * 5 + r0, jnp.int32)
                colc = pv - 40 * r0
            elif brk >= 16:
                rowc = jnp.full((16,), bi * 5 + r0, jnp.int32)
                colc = lane_i + (p0 - 40 * r0)
            else:
                ind = (lane_i >= brk).astype(jnp.int32)
                rowc = (bi * 5 + r0) + ind
                colc = lane_i + (p0 - 40 * r0) - 40 * ind
            ids = plsc.load_gather(idx_e, [rowc, colc])
            s = jnp.where(ids == 0, -99999.0, s)
            if g == NG - 1:
                s = jnp.where(lane_i >= 8, -jnp.inf, s)
            sv.append(s)
        mt = sv
        while len(mt) > 1:
            mt = [jnp.maximum(mt[k], mt[k + 1]) if k + 1 < len(mt) else mt[k]
                  for k in range(0, len(mt), 2)]
        bmax = bcast_last_max(mt[0])
        ev = [jnp.exp(s - bmax) for s in sv]
        while len(ev) > 1:
            ev = [ev[k] + ev[k + 1] if k + 1 < len(ev) else ev[k]
                  for k in range(0, len(ev), 2)]
        bsum = bcast_last_max(plsc.cumsum(ev[0]))
        bits = plsc.bitcast(bsum, jnp.int32)
        e_val = ((bits >> 23) & 0xFF) - 127
        mant = plsc.bitcast((bits & 0x7FFFFF) | 0x3F800000, _F32)
        tq = (mant - 1.0) / (mant + 1.0)
        t2 = tq * tq
        lnm = 2.0 * tq * (1.0 + t2 * (1.0 / 3.0 + t2 * (0.2 + t2 / 7.0)))
        lsum = e_val.astype(_F32) * LN2 + lnm
        off = bmax + lsum
        for g in range(NG):
            sg = sv[g] - off
            scores_v[pl.ds(bi * SW + g * 16, 16)] = sg

    issue(0, 0, rows_r0, rows_e0, sem0)

    def step(t, carry):
        issue(t, 1, rows_r1, rows_e1, sem1)
        drain(0, rows_r0, rows_e0, sem0)
        compute(t, 0, rows_r0, rows_e0)

        @pl.when(t + 1 < BPW)
        def _():
            issue(t + 1, 0, rows_r0, rows_e0, sem0)

        drain(1, rows_r1, rows_e1, sem1)
        compute(t, 1, rows_r1, rows_e1)
        log_softmax_b(t)
        return carry

    lax.fori_loop(0, BPW, step, 0)
    pltpu.sync_copy(scores_v, scores_hbm.at[pl.ds(base * SW, BPW * SW)])


def _score(rel_ids, ent_ids, out, emb):
    f = pl.kernel(
        _score_body,
        out_type=jax.ShapeDtypeStruct((B * SW,), _F32),
        mesh=_mesh(),
        scratch_types=[
            pltpu.VMEM((BPW * 5, 40), jnp.int32),
            pltpu.VMEM((BPW * 5, 40), jnp.int32),
            pltpu.VMEM((BPW, AE), _F32),
            pltpu.VMEM((96, D), _F32),
            pltpu.VMEM((96, D), _F32),
            pltpu.VMEM((104, D), _F32),
            pltpu.VMEM((104, D), _F32),
            pltpu.VMEM((BPW * SW,), _F32),
            pltpu.SemaphoreType.DMA,
            pltpu.SemaphoreType.DMA,
        ],
        compiler_params=pltpu.CompilerParams(needs_layout_passes=False),
    )
    return f(rel_ids, ent_ids, out, emb)


def kernel(prev_relation, current_entity, actions_id, queries, emb,
           W_ih, W_hh, b_ih, b_hh, W1, b1, W2, b2):
    del W_hh
    prev32 = prev_relation.astype(jnp.int32)
    cur32 = current_entity.astype(jnp.int32)
    qry32 = queries.astype(jnp.int32)
    act32 = actions_id.astype(jnp.int32)
    rel_ids = act32[:, :, 0]
    ent_ids = act32[:, :, 1]

    rel_e, cur_e, qry_e = _gather3(prev32, cur32, qry32, emb)
    out = _dense(rel_e, cur_e, qry_e, W_ih, b_ih, b_hh, W1, b1, W2, b2)
    logits = _score(rel_ids.reshape(B * 5, 40), ent_ids.reshape(B * 5, 40),
                    out, emb).reshape(B, SW)[:, :A]
    return (logits, rel_ids, ent_ids)
